# baseline (speedup 1.0000x reference)
import functools

import jax
import jax.numpy as jnp
from jax.experimental import pallas as pl
from jax.experimental.pallas import tpu as pltpu

N = 2048
RT = 256
NRT = N // RT
NMOL = 16


def _make_body(W):
    def body(bases_ref, attrs_ref, batch_ref, ee_ref, se_ref, aee_ref, ase_ref,
             refA_ref, refB_ref, refC_ref, refD_ref, refmu_ref,
             R_hbm, F_hbm, etot_ref, atomic_ref, rbuf, fbuf, sem):
        def make_copies(r, slot):
            base = pl.multiple_of(bases_ref[r], RT)
            cR = pltpu.make_async_copy(
                R_hbm.at[pl.ds(r * RT, RT), pl.ds(base, W)], rbuf.at[slot],
                sem.at[slot, 0])
            cF = pltpu.make_async_copy(
                F_hbm.at[pl.ds(r * RT, RT), pl.ds(base, W)], fbuf.at[slot],
                sem.at[slot, 1])
            return cR, cF

        pending = make_copies(0, 0)
        pending[0].start()
        pending[1].start()

        refB = refB_ref[...]
        Ae = refA_ref[...] * jnp.exp(refB * refmu_ref[...])
        refBn = -refB
        refC = refC_ref[...]
        refD = refD_ref[...]
        iota16 = jax.lax.broadcasted_iota(jnp.int32, (1, NMOL), 1)
        half = jnp.full((W, 1), 0.5, dtype=jnp.float32)
        dn_cols = (((1,), (1,)), ((), ()))
        dn_rows = (((0,), (0,)), ((), ()))

        etot_acc = ee_ref[...] + se_ref[...]

        for r in range(NRT):
            slot = r % 2
            if r + 1 < NRT:
                nxt = make_copies(r + 1, 1 - slot)
                nxt[0].start()
                nxt[1].start()
            pending[0].wait()
            pending[1].wait()
            if r + 1 < NRT:
                pending = nxt

            base = pl.multiple_of(bases_ref[r], RT)
            attrs_r = attrs_ref[r * RT:(r + 1) * RT, :]
            rowsAe = jnp.dot(attrs_r, Ae, preferred_element_type=jnp.float32)
            rowsBn = jnp.dot(attrs_r, refBn, preferred_element_type=jnp.float32)
            rowsC = jnp.dot(attrs_r, refC, preferred_element_type=jnp.float32)
            rowsD = jnp.dot(attrs_r, refD, preferred_element_type=jnp.float32)

            attrs_c = attrs_ref[pl.ds(base, W), :]
            batch_r = batch_ref[r * RT:(r + 1) * RT, :]
            batch_c = batch_ref[pl.ds(base, W), :]
            moh_r = (batch_r == iota16).astype(jnp.float32)
            moh_c = (batch_c == iota16).astype(jnp.float32)

            same = jax.lax.dot_general(moh_r, moh_c, dn_cols,
                                       preferred_element_type=jnp.float32)
            Aem = jax.lax.dot_general(rowsAe, attrs_c, dn_cols,
                                      preferred_element_type=jnp.float32)
            Bmn = jax.lax.dot_general(rowsBn, attrs_c, dn_cols,
                                      preferred_element_type=jnp.float32)
            Cm = jax.lax.dot_general(rowsC, attrs_c, dn_cols,
                                     preferred_element_type=jnp.float32)
            Dm = jax.lax.dot_general(rowsD, attrs_c, dn_cols,
                                     preferred_element_type=jnp.float32)

            Rb = rbuf[slot]
            Fb = fbuf[slot]
            rid = jax.lax.broadcasted_iota(jnp.int32, (RT, 1), 0) + r * RT
            cid = jax.lax.broadcasted_iota(jnp.int32, (1, W), 1) + base
            Feff = jnp.where(rid != cid, Fb * same, 0.0)

            r2 = Rb * Rb
            inv2 = 1.0 / r2
            inv4 = inv2 * inv2
            inv8 = inv4 * inv4
            e = (Aem * jnp.exp(Bmn * Rb) - (Cm * r2 + Dm) * inv8) * Feff

            partial = jnp.dot(e, half, preferred_element_type=jnp.float32)
            atomic_ref[r * RT:(r + 1) * RT, :] = (
                aee_ref[r * RT:(r + 1) * RT, :]
                + ase_ref[r * RT:(r + 1) * RT, :] + partial)
            etot_acc = etot_acc + jax.lax.dot_general(
                moh_r, partial, dn_rows, preferred_element_type=jnp.float32)

        etot_ref[...] = etot_acc

    return body


def _make_call(W):
    full = lambda shape: pl.BlockSpec(shape, lambda i, b: tuple(0 for _ in shape))
    tiny = full((4, 4))
    grid_spec = pltpu.PrefetchScalarGridSpec(
        num_scalar_prefetch=1,
        grid=(1,),
        in_specs=[
            full((N, 4)),
            full((N, 1)),
            full((NMOL, 1)),
            full((NMOL, 1)),
            full((N, 1)),
            full((N, 1)),
            tiny, tiny, tiny, tiny, tiny,
            pl.BlockSpec(memory_space=pl.ANY),
            pl.BlockSpec(memory_space=pl.ANY),
        ],
        out_specs=[
            full((NMOL, 1)),
            full((N, 1)),
        ],
        scratch_shapes=[
            pltpu.VMEM((2, RT, W), jnp.float32),
            pltpu.VMEM((2, RT, W), jnp.float32),
            pltpu.SemaphoreType.DMA((2, 2)),
        ],
    )
    return pl.pallas_call(
        _make_body(W),
        grid_spec=grid_spec,
        out_shape=[
            jax.ShapeDtypeStruct((NMOL, 1), jnp.float32),
            jax.ShapeDtypeStruct((N, 1), jnp.float32),
        ],
        compiler_params=pltpu.CompilerParams(
            dimension_semantics=("arbitrary",)),
    )


_FAST_W = 1024


def kernel(node_attrs, batch, R, F_cut, electric_energy, atomic_electric_energy,
           short_energy, atomic_short_energy, ref_A, ref_B, ref_C, ref_D, ref_mu):
    batch = batch.astype(jnp.int32)
    batch_col = batch.reshape(N, 1)

    m_lo = batch[::RT]
    m_hi = batch[RT - 1::RT]
    col_lo = jnp.searchsorted(batch, m_lo, side='left').astype(jnp.int32)
    col_hi = jnp.searchsorted(batch, m_hi, side='right').astype(jnp.int32)
    bases_fast = jnp.minimum((col_lo // RT) * RT, N - _FAST_W)
    fits = jnp.all(col_hi - bases_fast <= _FAST_W)
    bases_slow = jnp.zeros((NRT,), jnp.int32)

    ops = (node_attrs, batch_col, electric_energy, short_energy,
           atomic_electric_energy, atomic_short_energy,
           ref_A, ref_B, ref_C, ref_D, ref_mu, R, F_cut)

    etot, atomic = jax.lax.cond(
        fits,
        lambda a: _make_call(_FAST_W)(bases_fast, *a),
        lambda a: _make_call(N)(bases_slow, *a),
        ops)
    return (etot, atomic)


# speedup vs baseline: 1.6256x; 1.6256x over previous
import jax
import jax.numpy as jnp
from jax.experimental import pallas as pl
from jax.experimental.pallas import tpu as pltpu

N = 2048
RT = 256
NRT = N // RT
NMOL = 16
W = 1024


def _body(scal_ref, attrs_ref, batch_ref, ee_ref, se_ref, aee_ref, ase_ref,
          refA_ref, refB_ref, refC_ref, refD_ref, refmu_ref,
          R_hbm, F_hbm, etot_ref, atomic_ref, rbuf, fbuf, rbuf2, fbuf2, sem,
          sem2):
    def copies0(r, slot):
        base = pl.multiple_of(scal_ref[r], RT)
        cR = pltpu.make_async_copy(
            R_hbm.at[pl.ds(r * RT, RT), pl.ds(base, W)], rbuf.at[slot],
            sem.at[slot, 0])
        cF = pltpu.make_async_copy(
            F_hbm.at[pl.ds(r * RT, RT), pl.ds(base, W)], fbuf.at[slot],
            sem.at[slot, 1])
        return cR, cF

    def copies1(r):
        base = pl.multiple_of(scal_ref[16 + r], RT)
        cR = pltpu.make_async_copy(
            R_hbm.at[pl.ds(r * RT, RT), pl.ds(base, W)], rbuf2,
            sem2.at[0])
        cF = pltpu.make_async_copy(
            F_hbm.at[pl.ds(r * RT, RT), pl.ds(base, W)], fbuf2,
            sem2.at[1])
        return cR, cF

    pending = copies0(0, 0)
    pending[0].start()
    pending[1].start()

    refB = refB_ref[...]
    Ae = refA_ref[...] * jnp.exp(refB * refmu_ref[...])
    refBn = -refB
    refC = refC_ref[...]
    refD = refD_ref[...]
    iota16 = jax.lax.broadcasted_iota(jnp.int32, (1, NMOL), 1)
    half = jnp.full((W, 1), 0.5, dtype=jnp.float32)
    dn_cols = (((1,), (1,)), ((), ()))
    dn_rows = (((0,), (0,)), ((), ()))

    etot_ref[...] = ee_ref[...] + se_ref[...]

    for r in range(NRT):
        slot = r % 2
        if r + 1 < NRT:
            nxt = copies0(r + 1, 1 - slot)
            nxt[0].start()
            nxt[1].start()

        need2 = scal_ref[8 + r] > 0

        @pl.when(need2)
        def _():
            c1 = copies1(r)
            c1[0].start()
            c1[1].start()

        pending[0].wait()
        pending[1].wait()
        if r + 1 < NRT:
            pending = nxt

        attrs_r = attrs_ref[r * RT:(r + 1) * RT, :]
        rowsAe = jnp.dot(attrs_r, Ae, preferred_element_type=jnp.float32)
        rowsBn = jnp.dot(attrs_r, refBn, preferred_element_type=jnp.float32)
        rowsC = jnp.dot(attrs_r, refC, preferred_element_type=jnp.float32)
        rowsD = jnp.dot(attrs_r, refD, preferred_element_type=jnp.float32)

        batch_r = batch_ref[r * RT:(r + 1) * RT, :]
        moh_r = (batch_r == iota16).astype(jnp.float32)
        rid = jax.lax.broadcasted_iota(jnp.int32, (RT, 1), 0) + r * RT

        def window(base, Rb, Fb, extra_lo=None):
            attrs_c = attrs_ref[pl.ds(base, W), :]
            batch_c = batch_ref[pl.ds(base, W), :]
            moh_c = (batch_c == iota16).astype(jnp.float32)

            same = jax.lax.dot_general(moh_r, moh_c, dn_cols,
                                       preferred_element_type=jnp.float32)
            Aem = jax.lax.dot_general(rowsAe, attrs_c, dn_cols,
                                      preferred_element_type=jnp.float32)
            Bmn = jax.lax.dot_general(rowsBn, attrs_c, dn_cols,
                                      preferred_element_type=jnp.float32)
            Cm = jax.lax.dot_general(rowsC, attrs_c, dn_cols,
                                     preferred_element_type=jnp.float32)
            Dm = jax.lax.dot_general(rowsD, attrs_c, dn_cols,
                                     preferred_element_type=jnp.float32)

            cid = jax.lax.broadcasted_iota(jnp.int32, (1, W), 1) + base
            keep = rid != cid
            if extra_lo is not None:
                keep = keep & (cid >= extra_lo)
            Feff = jnp.where(keep, Fb * same, 0.0)

            r2 = Rb * Rb
            inv2 = 1.0 / r2
            inv4 = inv2 * inv2
            inv8 = inv4 * inv4
            e = (Aem * jnp.exp(Bmn * Rb) - (Cm * r2 + Dm) * inv8) * Feff
            return jnp.dot(e, half, preferred_element_type=jnp.float32)

        base0 = pl.multiple_of(scal_ref[r], RT)
        partial = window(base0, rbuf[slot], fbuf[slot])
        atomic_ref[r * RT:(r + 1) * RT, :] = (
            aee_ref[r * RT:(r + 1) * RT, :]
            + ase_ref[r * RT:(r + 1) * RT, :] + partial)
        etot_ref[...] = etot_ref[...] + jax.lax.dot_general(
            moh_r, partial, dn_rows, preferred_element_type=jnp.float32)

        @pl.when(need2)
        def _():
            c1 = copies1(r)
            c1[0].wait()
            c1[1].wait()
            base1 = pl.multiple_of(scal_ref[16 + r], RT)
            partial1 = window(base1, rbuf2[...], fbuf2[...], extra_lo=base0 + W)
            atomic_ref[r * RT:(r + 1) * RT, :] = (
                atomic_ref[r * RT:(r + 1) * RT, :] + partial1)
            etot_ref[...] = etot_ref[...] + jax.lax.dot_general(
                moh_r, partial1, dn_rows, preferred_element_type=jnp.float32)


def kernel(node_attrs, batch, R, F_cut, electric_energy, atomic_electric_energy,
           short_energy, atomic_short_energy, ref_A, ref_B, ref_C, ref_D, ref_mu):
    batch = batch.astype(jnp.int32)
    batch_col = batch.reshape(N, 1)

    m_lo = batch[::RT]
    m_hi = batch[RT - 1::RT]
    col_lo = jnp.sum(batch[None, :] < m_lo[:, None], axis=1).astype(jnp.int32)
    col_hi = jnp.sum(batch[None, :] <= m_hi[:, None], axis=1).astype(jnp.int32)
    base0 = jnp.minimum((col_lo // RT) * RT, N - W)
    need2 = (col_hi > base0 + W).astype(jnp.int32)
    base1 = jnp.minimum(base0 + W, N - W)
    scal = jnp.concatenate([base0, need2, base1])

    grid_spec = pltpu.PrefetchScalarGridSpec(
        num_scalar_prefetch=1,
        grid=(1,),
        in_specs=[
            pl.BlockSpec((N, 4), lambda i, s: (0, 0)),
            pl.BlockSpec((N, 1), lambda i, s: (0, 0)),
            pl.BlockSpec((NMOL, 1), lambda i, s: (0, 0)),
            pl.BlockSpec((NMOL, 1), lambda i, s: (0, 0)),
            pl.BlockSpec((N, 1), lambda i, s: (0, 0)),
            pl.BlockSpec((N, 1), lambda i, s: (0, 0)),
            pl.BlockSpec((4, 4), lambda i, s: (0, 0)),
            pl.BlockSpec((4, 4), lambda i, s: (0, 0)),
            pl.BlockSpec((4, 4), lambda i, s: (0, 0)),
            pl.BlockSpec((4, 4), lambda i, s: (0, 0)),
            pl.BlockSpec((4, 4), lambda i, s: (0, 0)),
            pl.BlockSpec(memory_space=pl.ANY),
            pl.BlockSpec(memory_space=pl.ANY),
        ],
        out_specs=[
            pl.BlockSpec((NMOL, 1), lambda i, s: (0, 0)),
            pl.BlockSpec((N, 1), lambda i, s: (0, 0)),
        ],
        scratch_shapes=[
            pltpu.VMEM((2, RT, W), jnp.float32),
            pltpu.VMEM((2, RT, W), jnp.float32),
            pltpu.VMEM((RT, W), jnp.float32),
            pltpu.VMEM((RT, W), jnp.float32),
            pltpu.SemaphoreType.DMA((2, 2)),
            pltpu.SemaphoreType.DMA((2,)),
        ],
    )
    etot, atomic = pl.pallas_call(
        _body,
        grid_spec=grid_spec,
        out_shape=[
            jax.ShapeDtypeStruct((NMOL, 1), jnp.float32),
            jax.ShapeDtypeStruct((N, 1), jnp.float32),
        ],
        compiler_params=pltpu.CompilerParams(
            dimension_semantics=("arbitrary",)),
    )(scal, node_attrs, batch_col, electric_energy, short_energy,
      atomic_electric_energy, atomic_short_energy,
      ref_A, ref_B, ref_C, ref_D, ref_mu, R, F_cut)
    return (etot, atomic)


# speedup vs baseline: 1.7061x; 1.0495x over previous
import jax
import jax.numpy as jnp
from jax.experimental import pallas as pl
from jax.experimental.pallas import tpu as pltpu

N = 2048
RT = 256
NRT = N // RT
NMOL = 16
W = 1024


def _bsearch(batch_s, target, right):
    lo = jnp.int32(0)
    hi = jnp.int32(N)
    for _ in range(11):
        cont = lo < hi
        mid = jnp.minimum((lo + hi) // 2, N - 1)
        v = batch_s[mid]
        go_right = (v <= target) if right else (v < target)
        new_lo = jnp.where(go_right, mid + 1, lo)
        new_hi = jnp.where(go_right, hi, mid)
        lo = jnp.where(cont, new_lo, lo)
        hi = jnp.where(cont, new_hi, hi)
    return lo


def _body(batch_s, attrs_ref, batch_ref, ee_ref, se_ref, aee_ref, ase_ref,
          refA_ref, refB_ref, refC_ref, refD_ref, refmu_ref,
          R_hbm, F_hbm, etot_ref, atomic_ref, rbuf, fbuf, rbuf2, fbuf2, sem,
          sem2):
    bands = []
    for r in range(NRT):
        m_lo = batch_s[r * RT]
        m_hi = batch_s[r * RT + RT - 1]
        lo0 = _bsearch(batch_s, m_lo, right=False)
        hi0 = _bsearch(batch_s, m_hi, right=True)
        base0 = pl.multiple_of(jnp.minimum((lo0 // RT) * RT, N - W), RT)
        need2 = hi0 > base0 + W
        base1 = pl.multiple_of(jnp.minimum(base0 + W, N - W), RT)
        bands.append((base0, need2, base1))

    def copies0(r, slot):
        base = bands[r][0]
        cR = pltpu.make_async_copy(
            R_hbm.at[pl.ds(r * RT, RT), pl.ds(base, W)], rbuf.at[slot],
            sem.at[slot, 0])
        cF = pltpu.make_async_copy(
            F_hbm.at[pl.ds(r * RT, RT), pl.ds(base, W)], fbuf.at[slot],
            sem.at[slot, 1])
        return cR, cF

    def copies1(r):
        base = bands[r][2]
        cR = pltpu.make_async_copy(
            R_hbm.at[pl.ds(r * RT, RT), pl.ds(base, W)], rbuf2,
            sem2.at[0])
        cF = pltpu.make_async_copy(
            F_hbm.at[pl.ds(r * RT, RT), pl.ds(base, W)], fbuf2,
            sem2.at[1])
        return cR, cF

    pending = copies0(0, 0)
    pending[0].start()
    pending[1].start()

    refB = refB_ref[...]
    Ae = refA_ref[...] * jnp.exp(refB * refmu_ref[...])
    refBn = -refB
    refC = refC_ref[...]
    refD = refD_ref[...]
    iota16 = jax.lax.broadcasted_iota(jnp.int32, (1, NMOL), 1)
    half = jnp.full((W, 1), 0.5, dtype=jnp.float32)
    half4 = jnp.full((4, 1), 0.5, dtype=jnp.float32)
    dn_cols = (((1,), (1,)), ((), ()))
    dn_rows = (((0,), (0,)), ((), ()))
    dn_mm = (((1,), (0,)), ((), ()))

    etot_ref[...] = ee_ref[...] + se_ref[...]

    for r in range(NRT):
        slot = r % 2
        if r + 1 < NRT:
            nxt = copies0(r + 1, 1 - slot)
            nxt[0].start()
            nxt[1].start()

        need2 = bands[r][1]

        @pl.when(need2)
        def _():
            c1 = copies1(r)
            c1[0].start()
            c1[1].start()

        pending[0].wait()
        pending[1].wait()
        if r + 1 < NRT:
            pending = nxt

        attrs_r = attrs_ref[r * RT:(r + 1) * RT, :]
        rowsAe = jnp.dot(attrs_r, Ae, preferred_element_type=jnp.float32)
        rowsBn = jnp.dot(attrs_r, refBn, preferred_element_type=jnp.float32)
        rowsC = jnp.dot(attrs_r, refC, preferred_element_type=jnp.float32)
        rowsD = jnp.dot(attrs_r, refD, preferred_element_type=jnp.float32)

        batch_r = batch_ref[r * RT:(r + 1) * RT, :]
        moh_r = (batch_r == iota16).astype(jnp.float32)
        rid = jax.lax.broadcasted_iota(jnp.int32, (RT, 1), 0) + r * RT

        def window(base, Rb, Fb, extra_lo=None):
            attrs_c = attrs_ref[pl.ds(base, W), :]
            batch_c = batch_ref[pl.ds(base, W), :]
            moh_c = (batch_c == iota16).astype(jnp.float32)

            same = jax.lax.dot_general(moh_r, moh_c, dn_cols,
                                       preferred_element_type=jnp.float32)
            Aem = jax.lax.dot_general(rowsAe, attrs_c, dn_cols,
                                      preferred_element_type=jnp.float32)
            Bmn = jax.lax.dot_general(rowsBn, attrs_c, dn_cols,
                                      preferred_element_type=jnp.float32)

            cid = jax.lax.broadcasted_iota(jnp.int32, (1, W), 1) + base
            keep = rid != cid
            if extra_lo is not None:
                keep = keep & (cid >= extra_lo)
            Feff = jnp.where(keep, Fb * same, 0.0)

            r2 = Rb * Rb
            inv2 = 1.0 / r2
            i4 = inv2 * inv2
            inv6 = i4 * inv2
            XC = inv6 * Feff
            XD = XC * inv2
            e1 = Aem * jnp.exp(Bmn * Rb) * Feff
            partial1 = jnp.dot(e1, half, preferred_element_type=jnp.float32)

            GC = jax.lax.dot_general(XC, attrs_c, dn_mm,
                                     preferred_element_type=jnp.float32)
            GD = jax.lax.dot_general(XD, attrs_c, dn_mm,
                                     preferred_element_type=jnp.float32)
            S = rowsC * GC + rowsD * GD
            return partial1 - jnp.dot(S, half4,
                                      preferred_element_type=jnp.float32)

        base0 = bands[r][0]
        partial = window(base0, rbuf[slot], fbuf[slot])
        atomic_ref[r * RT:(r + 1) * RT, :] = (
            aee_ref[r * RT:(r + 1) * RT, :]
            + ase_ref[r * RT:(r + 1) * RT, :] + partial)
        etot_ref[...] = etot_ref[...] + jax.lax.dot_general(
            moh_r, partial, dn_rows, preferred_element_type=jnp.float32)

        @pl.when(need2)
        def _():
            c1 = copies1(r)
            c1[0].wait()
            c1[1].wait()
            partial1 = window(bands[r][2], rbuf2[...], fbuf2[...],
                              extra_lo=base0 + W)
            atomic_ref[r * RT:(r + 1) * RT, :] = (
                atomic_ref[r * RT:(r + 1) * RT, :] + partial1)
            etot_ref[...] = etot_ref[...] + jax.lax.dot_general(
                moh_r, partial1, dn_rows, preferred_element_type=jnp.float32)


def kernel(node_attrs, batch, R, F_cut, electric_energy, atomic_electric_energy,
           short_energy, atomic_short_energy, ref_A, ref_B, ref_C, ref_D, ref_mu):
    batch = batch.astype(jnp.int32)
    batch_col = batch.reshape(N, 1)

    grid_spec = pltpu.PrefetchScalarGridSpec(
        num_scalar_prefetch=1,
        grid=(1,),
        in_specs=[
            pl.BlockSpec((N, 4), lambda i, s: (0, 0)),
            pl.BlockSpec((N, 1), lambda i, s: (0, 0)),
            pl.BlockSpec((NMOL, 1), lambda i, s: (0, 0)),
            pl.BlockSpec((NMOL, 1), lambda i, s: (0, 0)),
            pl.BlockSpec((N, 1), lambda i, s: (0, 0)),
            pl.BlockSpec((N, 1), lambda i, s: (0, 0)),
            pl.BlockSpec((4, 4), lambda i, s: (0, 0)),
            pl.BlockSpec((4, 4), lambda i, s: (0, 0)),
            pl.BlockSpec((4, 4), lambda i, s: (0, 0)),
            pl.BlockSpec((4, 4), lambda i, s: (0, 0)),
            pl.BlockSpec((4, 4), lambda i, s: (0, 0)),
            pl.BlockSpec(memory_space=pl.ANY),
            pl.BlockSpec(memory_space=pl.ANY),
        ],
        out_specs=[
            pl.BlockSpec((NMOL, 1), lambda i, s: (0, 0)),
            pl.BlockSpec((N, 1), lambda i, s: (0, 0)),
        ],
        scratch_shapes=[
            pltpu.VMEM((2, RT, W), jnp.float32),
            pltpu.VMEM((2, RT, W), jnp.float32),
            pltpu.VMEM((RT, W), jnp.float32),
            pltpu.VMEM((RT, W), jnp.float32),
            pltpu.SemaphoreType.DMA((2, 2)),
            pltpu.SemaphoreType.DMA((2,)),
        ],
    )
    etot, atomic = pl.pallas_call(
        _body,
        grid_spec=grid_spec,
        out_shape=[
            jax.ShapeDtypeStruct((NMOL, 1), jnp.float32),
            jax.ShapeDtypeStruct((N, 1), jnp.float32),
        ],
        compiler_params=pltpu.CompilerParams(
            dimension_semantics=("arbitrary",)),
    )(batch, node_attrs, batch_col, electric_energy, short_energy,
      atomic_electric_energy, atomic_short_energy,
      ref_A, ref_B, ref_C, ref_D, ref_mu, R, F_cut)
    return (etot, atomic)


# speedup vs baseline: 1.9657x; 1.1522x over previous
import jax
import jax.numpy as jnp
from jax.experimental import pallas as pl
from jax.experimental.pallas import tpu as pltpu

N = 2048
RT = 256
NRT = N // RT
NMOL = 16
W = 1024


def _bsearch(batch_s, target, right):
    lo = jnp.int32(0)
    hi = jnp.int32(N)
    for _ in range(11):
        cont = lo < hi
        mid = jnp.minimum((lo + hi) // 2, N - 1)
        v = batch_s[mid]
        go_right = (v <= target) if right else (v < target)
        new_lo = jnp.where(go_right, mid + 1, lo)
        new_hi = jnp.where(go_right, hi, mid)
        lo = jnp.where(cont, new_lo, lo)
        hi = jnp.where(cont, new_hi, hi)
    return lo


def _body(batch_s, attrs_ref, moh_ref, vec_ref, refs_ref,
          R_hbm, F_hbm, etot_ref, atomic_ref, rbuf, fbuf, rbuf2, fbuf2, sem,
          sem2):
    bands = []
    for r in range(NRT):
        m_lo = batch_s[r * RT]
        m_hi = batch_s[r * RT + RT - 1]
        lo0 = _bsearch(batch_s, m_lo, right=False)
        hi0 = _bsearch(batch_s, m_hi, right=True)
        base0 = pl.multiple_of(jnp.minimum((lo0 // RT) * RT, N - W), RT)
        need2 = hi0 > base0 + W
        base1 = pl.multiple_of(jnp.minimum(base0 + W, N - W), RT)
        bands.append((base0, need2, base1))

    def copies0(r, slot):
        base = bands[r][0]
        cR = pltpu.make_async_copy(
            R_hbm.at[pl.ds(r * RT, RT), pl.ds(base, W)], rbuf.at[slot],
            sem.at[slot, 0])
        cF = pltpu.make_async_copy(
            F_hbm.at[pl.ds(r * RT, RT), pl.ds(base, W)], fbuf.at[slot],
            sem.at[slot, 1])
        return cR, cF

    def copies1(r):
        base = bands[r][2]
        cR = pltpu.make_async_copy(
            R_hbm.at[pl.ds(r * RT, RT), pl.ds(base, W)], rbuf2,
            sem2.at[0])
        cF = pltpu.make_async_copy(
            F_hbm.at[pl.ds(r * RT, RT), pl.ds(base, W)], fbuf2,
            sem2.at[1])
        return cR, cF

    pending = copies0(0, 0)
    pending[0].start()
    pending[1].start()

    refB = refs_ref[:, 4:8]
    Ae = refs_ref[:, 0:4] * jnp.exp(refB * refs_ref[:, 16:20])
    refBn = -refB
    refC = refs_ref[:, 8:12]
    refD = refs_ref[:, 12:16]
    half = jnp.full((W, 1), 0.5, dtype=jnp.float32)
    half4 = jnp.full((4, 1), 0.5, dtype=jnp.float32)
    dn_cols = (((1,), (1,)), ((), ()))
    dn_rows = (((0,), (0,)), ((), ()))
    dn_mm = (((1,), (0,)), ((), ()))

    etot_ref[...] = vec_ref[N:N + NMOL, :]

    for r in range(NRT):
        slot = r % 2
        if r + 1 < NRT:
            nxt = copies0(r + 1, 1 - slot)
            nxt[0].start()
            nxt[1].start()

        need2 = bands[r][1]

        @pl.when(need2)
        def _():
            c1 = copies1(r)
            c1[0].start()
            c1[1].start()

        pending[0].wait()
        pending[1].wait()
        if r + 1 < NRT:
            pending = nxt

        attrs_r = attrs_ref[r * RT:(r + 1) * RT, :]
        rowsAe = jnp.dot(attrs_r, Ae, preferred_element_type=jnp.float32)
        rowsBn = jnp.dot(attrs_r, refBn, preferred_element_type=jnp.float32)
        rowsC = jnp.dot(attrs_r, refC, preferred_element_type=jnp.float32)
        rowsD = jnp.dot(attrs_r, refD, preferred_element_type=jnp.float32)

        moh_r = moh_ref[r * RT:(r + 1) * RT, :]
        rid = jax.lax.broadcasted_iota(jnp.int32, (RT, 1), 0) + r * RT

        def window(base, Rb, Fb, extra_lo=None):
            attrs_c = attrs_ref[pl.ds(base, W), :]
            moh_c = moh_ref[pl.ds(base, W), :]

            same = jax.lax.dot_general(moh_r, moh_c, dn_cols,
                                       preferred_element_type=jnp.float32)
            Aem = jax.lax.dot_general(rowsAe, attrs_c, dn_cols,
                                      preferred_element_type=jnp.float32)
            Bmn = jax.lax.dot_general(rowsBn, attrs_c, dn_cols,
                                      preferred_element_type=jnp.float32)

            cid = jax.lax.broadcasted_iota(jnp.int32, (1, W), 1) + base
            keep = rid != cid
            if extra_lo is not None:
                keep = keep & (cid >= extra_lo)
            Feff = jnp.where(keep, Fb * same, 0.0)

            r2 = Rb * Rb
            inv2 = 1.0 / r2
            i4 = inv2 * inv2
            inv6 = i4 * inv2
            XC = inv6 * Feff
            XD = XC * inv2
            e1 = Aem * jnp.exp(Bmn * Rb) * Feff
            partial1 = jnp.dot(e1, half, preferred_element_type=jnp.float32)

            GC = jax.lax.dot_general(XC, attrs_c, dn_mm,
                                     preferred_element_type=jnp.float32)
            GD = jax.lax.dot_general(XD, attrs_c, dn_mm,
                                     preferred_element_type=jnp.float32)
            S = rowsC * GC + rowsD * GD
            return partial1 - jnp.dot(S, half4,
                                      preferred_element_type=jnp.float32)

        base0 = bands[r][0]
        partial = window(base0, rbuf[slot], fbuf[slot])
        atomic_ref[r * RT:(r + 1) * RT, :] = (
            vec_ref[r * RT:(r + 1) * RT, :] + partial)
        etot_ref[...] = etot_ref[...] + jax.lax.dot_general(
            moh_r, partial, dn_rows, preferred_element_type=jnp.float32)

        @pl.when(need2)
        def _():
            c1 = copies1(r)
            c1[0].wait()
            c1[1].wait()
            partial1 = window(bands[r][2], rbuf2[...], fbuf2[...],
                              extra_lo=base0 + W)
            atomic_ref[r * RT:(r + 1) * RT, :] = (
                atomic_ref[r * RT:(r + 1) * RT, :] + partial1)
            etot_ref[...] = etot_ref[...] + jax.lax.dot_general(
                moh_r, partial1, dn_rows, preferred_element_type=jnp.float32)


def kernel(node_attrs, batch, R, F_cut, electric_energy, atomic_electric_energy,
           short_energy, atomic_short_energy, ref_A, ref_B, ref_C, ref_D, ref_mu):
    batch = batch.astype(jnp.int32)
    vec = jnp.concatenate(
        [atomic_electric_energy + atomic_short_energy,
         electric_energy + short_energy], axis=0)
    refs = jnp.concatenate([ref_A, ref_B, ref_C, ref_D, ref_mu], axis=1)
    moh = jax.nn.one_hot(batch, NMOL, dtype=jnp.float32)

    grid_spec = pltpu.PrefetchScalarGridSpec(
        num_scalar_prefetch=1,
        grid=(1,),
        in_specs=[
            pl.BlockSpec((N, 4), lambda i, s: (0, 0)),
            pl.BlockSpec((N, NMOL), lambda i, s: (0, 0)),
            pl.BlockSpec((N + NMOL, 1), lambda i, s: (0, 0)),
            pl.BlockSpec((4, 20), lambda i, s: (0, 0)),
            pl.BlockSpec(memory_space=pl.ANY),
            pl.BlockSpec(memory_space=pl.ANY),
        ],
        out_specs=[
            pl.BlockSpec((NMOL, 1), lambda i, s: (0, 0)),
            pl.BlockSpec((N, 1), lambda i, s: (0, 0)),
        ],
        scratch_shapes=[
            pltpu.VMEM((2, RT, W), jnp.float32),
            pltpu.VMEM((2, RT, W), jnp.float32),
            pltpu.VMEM((RT, W), jnp.float32),
            pltpu.VMEM((RT, W), jnp.float32),
            pltpu.SemaphoreType.DMA((2, 2)),
            pltpu.SemaphoreType.DMA((2,)),
        ],
    )
    etot, atomic = pl.pallas_call(
        _body,
        grid_spec=grid_spec,
        out_shape=[
            jax.ShapeDtypeStruct((NMOL, 1), jnp.float32),
            jax.ShapeDtypeStruct((N, 1), jnp.float32),
        ],
        compiler_params=pltpu.CompilerParams(
            dimension_semantics=("arbitrary",)),
    )(batch, node_attrs, moh, vec, refs, R, F_cut)
    return (etot, atomic)
